# Initial kernel scaffold; baseline (speedup 1.0000x reference)
import jax
import jax.numpy as jnp
from jax.experimental import pallas as pl


def kernel(x_nodes, x_edges, edge_index, We1, be1, We2, be2, Wn1, bn1, Wn2, bn2):
    raise NotImplementedError("write your pallas kernel here")


# speedup vs baseline: 2.7035x; 2.7035x over previous
import functools

import jax
import jax.numpy as jnp
from jax import lax
from jax.experimental import pallas as pl
from jax.experimental.pallas import tpu as pltpu
from jax.experimental.pallas import tpu_sc as plsc

D = 128
NC, NS = 2, 16
NW = NC * NS
CHUNK = 80


def _silu(x):
    return x * (1.0 / (1.0 + jnp.exp(-x)))


def _pq_body(x_ref, ws_ref, wr_ref, p_ref, q_ref):
    x = x_ref[...]
    p_ref[...] = jnp.dot(x, ws_ref[...], preferred_element_type=jnp.float32)
    q_ref[...] = jnp.dot(x, wr_ref[...], preferred_element_type=jnp.float32)


def _compute_pq(x_nodes, Ws, Wr, nb):
    n = x_nodes.shape[0]
    return pl.pallas_call(
        _pq_body,
        grid=(n // nb,),
        in_specs=[
            pl.BlockSpec((nb, D), lambda i: (i, 0)),
            pl.BlockSpec((D, D), lambda i: (0, 0)),
            pl.BlockSpec((D, D), lambda i: (0, 0)),
        ],
        out_specs=[
            pl.BlockSpec((nb, D), lambda i: (i, 0)),
            pl.BlockSpec((nb, D), lambda i: (i, 0)),
        ],
        out_shape=[jax.ShapeDtypeStruct((n, D), jnp.float32)] * 2,
    )(x_nodes, Ws, Wr)


def _edge_body(g_ref, xe_ref, we_ref, b1_ref, w2_ref, b2_ref, m_ref):
    pre = (g_ref[...]
           + jnp.dot(xe_ref[...], we_ref[...], preferred_element_type=jnp.float32)
           + b1_ref[...])
    h = _silu(pre)
    m_ref[...] = _silu(
        jnp.dot(h, w2_ref[...], preferred_element_type=jnp.float32) + b2_ref[...])


def _edge_mlp(G, x_edges, We, b1, W2, b2, eb):
    e, de = x_edges.shape
    return pl.pallas_call(
        _edge_body,
        grid=(e // eb,),
        in_specs=[
            pl.BlockSpec((eb, D), lambda i: (i, 0)),
            pl.BlockSpec((eb, de), lambda i: (i, 0)),
            pl.BlockSpec((de, D), lambda i: (0, 0)),
            pl.BlockSpec((1, D), lambda i: (0, 0)),
            pl.BlockSpec((D, D), lambda i: (0, 0)),
            pl.BlockSpec((1, D), lambda i: (0, 0)),
        ],
        out_specs=pl.BlockSpec((eb, D), lambda i: (i, 0)),
        out_shape=jax.ShapeDtypeStruct((e, D), jnp.float32),
    )(G, x_edges, We, b1, W2, b2)


def _node_body(x_ref, a_ref, w1x_ref, w1a_ref, b1_ref, w2_ref, b2_ref, o_ref):
    a = a_ref[0] + a_ref[1]
    u = _silu(jnp.dot(x_ref[...], w1x_ref[...], preferred_element_type=jnp.float32)
              + jnp.dot(a, w1a_ref[...], preferred_element_type=jnp.float32)
              + b1_ref[...])
    o_ref[...] = (jnp.dot(u, w2_ref[...], preferred_element_type=jnp.float32)
                  + b2_ref[...])


def _node_net(x_nodes, A, W1x, W1a, b1, W2, b2, nb):
    n = x_nodes.shape[0]
    return pl.pallas_call(
        _node_body,
        grid=(n // nb,),
        in_specs=[
            pl.BlockSpec((nb, D), lambda i: (i, 0)),
            pl.BlockSpec((NC, nb, D), lambda i: (0, i, 0)),
            pl.BlockSpec((D, D), lambda i: (0, 0)),
            pl.BlockSpec((D, D), lambda i: (0, 0)),
            pl.BlockSpec((1, D), lambda i: (0, 0)),
            pl.BlockSpec((D, D), lambda i: (0, 0)),
            pl.BlockSpec((1, D), lambda i: (0, 0)),
        ],
        out_specs=pl.BlockSpec((nb, D), lambda i: (i, 0)),
        out_shape=jax.ShapeDtypeStruct((n, D), jnp.float32),
    )(x_nodes, A, W1x, W1a, b1, W2, b2)


def _sc_gather(P, Q, sender, receiver):
    n_edges = sender.shape[0]
    epw = n_edges // NW
    nchunk = epw // CHUNK
    mesh = plsc.VectorSubcoreMesh(core_axis_name="c", subcore_axis_name="s",
                                  num_cores=NC, num_subcores=NS)

    @functools.partial(
        pl.kernel,
        out_type=jax.ShapeDtypeStruct((n_edges, D), jnp.float32),
        mesh=mesh,
        scratch_types=[
            pltpu.VMEM((CHUNK,), jnp.int32),
            pltpu.VMEM((CHUNK,), jnp.int32),
            pltpu.VMEM((CHUNK, D), jnp.float32),
            pltpu.VMEM((CHUNK, D), jnp.float32),
            pltpu.SemaphoreType.DMA,
            pltpu.SemaphoreType.DMA,
        ],
    )
    def k(p_hbm, q_hbm, s_hbm, r_hbm, out_hbm, idx_s, idx_r, bufp, bufq, sem0, sem1):
        wid = lax.axis_index("s") * NC + lax.axis_index("c")
        base = wid * epw

        def chunk_body(c, carry):
            off = base + c * CHUNK
            pltpu.sync_copy(s_hbm.at[pl.ds(off, CHUNK)], idx_s)
            pltpu.sync_copy(r_hbm.at[pl.ds(off, CHUNK)], idx_r)
            cp = pltpu.async_copy(p_hbm.at[idx_s], bufp, sem0)
            cq = pltpu.async_copy(q_hbm.at[idx_r], bufq, sem1)
            cp.wait()
            cq.wait()

            def add_row(e, cc):
                for j in range(D // 16):
                    sl = pl.ds(j * 16, 16)
                    bufp[e, sl] = bufp[e, sl] + bufq[e, sl]
                return cc

            lax.fori_loop(0, CHUNK, add_row, 0)
            pltpu.sync_copy(bufp, out_hbm.at[pl.ds(off, CHUNK)])
            return carry

        lax.fori_loop(0, nchunk, chunk_body, 0)

    return k(P, Q, sender, receiver)


def _sc_scatter(M, receiver, n_nodes):
    n_edges = receiver.shape[0]
    epw = n_edges // NW
    nchunk = epw // CHUNK
    rows_per_sub = (n_nodes // NS) // 8 * 8
    tail_off = NS * rows_per_sub
    tail_rows = n_nodes - tail_off
    zr = 16
    mesh = plsc.VectorSubcoreMesh(core_axis_name="c", subcore_axis_name="s",
                                  num_cores=NC, num_subcores=NS)

    @functools.partial(
        pl.kernel,
        out_type=jax.ShapeDtypeStruct((NC, n_nodes, D), jnp.float32),
        mesh=mesh,
        scratch_types=[
            pltpu.VMEM((CHUNK,), jnp.int32),
            pltpu.VMEM((CHUNK, D), jnp.float32),
            pltpu.VMEM((zr, D), jnp.float32),
            pltpu.VMEM_SHARED((n_nodes, D), jnp.float32),
        ],
    )
    def k(m_hbm, r_hbm, out_hbm, idx, bufm, zbuf, acc):
        cid = lax.axis_index("c")
        sid = lax.axis_index("s")
        wid = sid * NC + cid

        zero = jnp.zeros((16,), jnp.float32)

        def zrow(e, cc):
            for j in range(D // 16):
                zbuf[e, pl.ds(j * 16, 16)] = zero
            return cc

        lax.fori_loop(0, zr, zrow, 0)

        def zcopy(kk, cc):
            pltpu.sync_copy(zbuf, acc.at[pl.ds(sid * rows_per_sub + kk * zr, zr)])
            return cc

        lax.fori_loop(0, rows_per_sub // zr, zcopy, 0)

        @pl.when(sid == NS - 1)
        def _():
            pltpu.sync_copy(zbuf, acc.at[pl.ds(tail_off, tail_rows)])

        plsc.subcore_barrier()

        base = wid * epw

        def chunk_body(c, carry):
            off = base + c * CHUNK
            pltpu.sync_copy(r_hbm.at[pl.ds(off, CHUNK)], idx)
            pltpu.sync_copy(m_hbm.at[pl.ds(off, CHUNK)], bufm)
            pltpu.sync_copy(bufm, acc.at[idx], add=True)
            return carry

        lax.fori_loop(0, nchunk, chunk_body, 0)
        plsc.subcore_barrier()
        pltpu.sync_copy(acc.at[pl.ds(sid * rows_per_sub, rows_per_sub)],
                        out_hbm.at[cid, pl.ds(sid * rows_per_sub, rows_per_sub)])

        @pl.when(sid == NS - 1)
        def _():
            pltpu.sync_copy(acc.at[pl.ds(tail_off, tail_rows)],
                            out_hbm.at[cid, pl.ds(tail_off, tail_rows)])

    return k(M, receiver)


def kernel(x_nodes, x_edges, edge_index, We1, be1, We2, be2, Wn1, bn1, Wn2, bn2):
    n_nodes = x_nodes.shape[0]
    sender = edge_index[0].astype(jnp.int32)
    receiver = edge_index[1].astype(jnp.int32)
    Ws, Wr, We = We1[:D], We1[D:2 * D], We1[2 * D:]
    b1e = be1.reshape(1, D)
    b2e = be2.reshape(1, D)
    b1n = bn1.reshape(1, D)
    b2n = bn2.reshape(1, D)

    P, Q = _compute_pq(x_nodes, Ws, Wr, nb=2000)
    G = _sc_gather(P, Q, sender, receiver)
    M = _edge_mlp(G, x_edges, We, b1e, We2, b2e, eb=2000)
    A = _sc_scatter(M, receiver, n_nodes)
    return _node_net(x_nodes, A, Wn1[:D], Wn1[D:], b1n, Wn2, b2n, nb=2000)


# speedup vs baseline: 3.2970x; 1.2195x over previous
import functools

import jax
import jax.numpy as jnp
from jax import lax
from jax.experimental import pallas as pl
from jax.experimental.pallas import tpu as pltpu
from jax.experimental.pallas import tpu_sc as plsc

D = 128
NC, NS = 2, 16
NW = NC * NS
CHUNK = 80


def _silu(x):
    return x * (1.0 / (1.0 + jnp.exp(-x)))


def _pq_body(x_ref, ws_ref, wr_ref, p_ref, q_ref):
    x = x_ref[...]
    p_ref[...] = jnp.dot(x, ws_ref[...], preferred_element_type=jnp.float32)
    q_ref[...] = jnp.dot(x, wr_ref[...], preferred_element_type=jnp.float32)


def _compute_pq(x_nodes, Ws, Wr, nb):
    n = x_nodes.shape[0]
    return pl.pallas_call(
        _pq_body,
        grid=(n // nb,),
        in_specs=[
            pl.BlockSpec((nb, D), lambda i: (i, 0)),
            pl.BlockSpec((D, D), lambda i: (0, 0)),
            pl.BlockSpec((D, D), lambda i: (0, 0)),
        ],
        out_specs=[
            pl.BlockSpec((nb, D), lambda i: (i, 0)),
            pl.BlockSpec((nb, D), lambda i: (i, 0)),
        ],
        out_shape=[jax.ShapeDtypeStruct((n, D), jnp.float32)] * 2,
    )(x_nodes, Ws, Wr)


def _edge_body(g_ref, xe_ref, we_ref, b1_ref, w2_ref, b2_ref, m_ref):
    pre = (g_ref[...]
           + jnp.dot(xe_ref[...], we_ref[...], preferred_element_type=jnp.float32)
           + b1_ref[...])
    h = _silu(pre)
    m_ref[...] = _silu(
        jnp.dot(h, w2_ref[...], preferred_element_type=jnp.float32) + b2_ref[...])


def _edge_mlp(G, x_edges, We, b1, W2, b2, eb):
    e, de = x_edges.shape
    return pl.pallas_call(
        _edge_body,
        grid=(e // eb,),
        in_specs=[
            pl.BlockSpec((eb, D), lambda i: (i, 0)),
            pl.BlockSpec((eb, de), lambda i: (i, 0)),
            pl.BlockSpec((de, D), lambda i: (0, 0)),
            pl.BlockSpec((1, D), lambda i: (0, 0)),
            pl.BlockSpec((D, D), lambda i: (0, 0)),
            pl.BlockSpec((1, D), lambda i: (0, 0)),
        ],
        out_specs=pl.BlockSpec((eb, D), lambda i: (i, 0)),
        out_shape=jax.ShapeDtypeStruct((e, D), jnp.float32),
    )(G, x_edges, We, b1, W2, b2)


def _node_body(x_ref, a_ref, w1x_ref, w1a_ref, b1_ref, w2_ref, b2_ref, o_ref):
    a = a_ref[0] + a_ref[1]
    u = _silu(jnp.dot(x_ref[...], w1x_ref[...], preferred_element_type=jnp.float32)
              + jnp.dot(a, w1a_ref[...], preferred_element_type=jnp.float32)
              + b1_ref[...])
    o_ref[...] = (jnp.dot(u, w2_ref[...], preferred_element_type=jnp.float32)
                  + b2_ref[...])


def _node_net(x_nodes, A, W1x, W1a, b1, W2, b2, nb):
    n = x_nodes.shape[0]
    return pl.pallas_call(
        _node_body,
        grid=(n // nb,),
        in_specs=[
            pl.BlockSpec((nb, D), lambda i: (i, 0)),
            pl.BlockSpec((NC, nb, D), lambda i: (0, i, 0)),
            pl.BlockSpec((D, D), lambda i: (0, 0)),
            pl.BlockSpec((D, D), lambda i: (0, 0)),
            pl.BlockSpec((1, D), lambda i: (0, 0)),
            pl.BlockSpec((D, D), lambda i: (0, 0)),
            pl.BlockSpec((1, D), lambda i: (0, 0)),
        ],
        out_specs=pl.BlockSpec((nb, D), lambda i: (i, 0)),
        out_shape=jax.ShapeDtypeStruct((n, D), jnp.float32),
    )(x_nodes, A, W1x, W1a, b1, W2, b2)


def _sc_gather(P, Q, sender3d, receiver3d):
    nchunk = sender3d.shape[1]
    epw = nchunk * CHUNK
    n_edges = NW * epw
    mesh = plsc.VectorSubcoreMesh(core_axis_name="c", subcore_axis_name="s",
                                  num_cores=NC, num_subcores=NS)

    @functools.partial(
        pl.kernel,
        out_type=jax.ShapeDtypeStruct((n_edges, D), jnp.float32),
        mesh=mesh,
        scratch_types=[
            pltpu.VMEM((nchunk, CHUNK), jnp.int32),
            pltpu.VMEM((nchunk, CHUNK), jnp.int32),
            pltpu.VMEM((2, CHUNK, D), jnp.float32),
            pltpu.VMEM((2, CHUNK, D), jnp.float32),
            pltpu.VMEM((2, CHUNK, D), jnp.float32),
            pltpu.SemaphoreType.DMA, pltpu.SemaphoreType.DMA,
            pltpu.SemaphoreType.DMA, pltpu.SemaphoreType.DMA,
            pltpu.SemaphoreType.DMA, pltpu.SemaphoreType.DMA,
        ],
    )
    def k(p_hbm, q_hbm, s_hbm, r_hbm, out_hbm,
          idxs, idxr, bufp, bufq, obuf, sp0, sp1, sq0, sq1, so0, so1):
        sp, sq, so = (sp0, sp1), (sq0, sq1), (so0, so1)
        wid = lax.axis_index("s") * NC + lax.axis_index("c")
        base = wid * epw

        pltpu.sync_copy(s_hbm.at[wid], idxs)
        pltpu.sync_copy(r_hbm.at[wid], idxr)

        def issue(c, b):
            pltpu.async_copy(p_hbm.at[idxs.at[c]], bufp.at[b], sp[b])
            pltpu.async_copy(q_hbm.at[idxr.at[c]], bufq.at[b], sq[b])

        def process(c, b, wait_prev_write):
            pltpu.make_async_copy(p_hbm.at[idxs.at[c]], bufp.at[b], sp[b]).wait()
            pltpu.make_async_copy(q_hbm.at[idxr.at[c]], bufq.at[b], sq[b]).wait()

            @pl.when(wait_prev_write)
            def _():
                pltpu.make_async_copy(
                    obuf.at[b], out_hbm.at[pl.ds(base, CHUNK)], so[b]).wait()

            def add_row(e, cc):
                for j in range(D // 16):
                    sl = pl.ds(j * 16, 16)
                    obuf[b, e, sl] = bufp[b, e, sl] + bufq[b, e, sl]
                return cc

            lax.fori_loop(0, CHUNK, add_row, 0, unroll=2)
            off = base + c * CHUNK
            pltpu.async_copy(obuf.at[b], out_hbm.at[pl.ds(off, CHUNK)], so[b])

        issue(0, 0)
        issue(1, 1)

        def pair_body(c2, carry):
            c0 = 2 * c2
            process(c0, 0, c2 > 0)
            issue(c0 + 2, 0)
            process(c0 + 1, 1, c2 > 0)

            @pl.when(c0 + 3 < nchunk)
            def _():
                issue(c0 + 3, 1)

            return carry

        lax.fori_loop(0, (nchunk - 1) // 2, pair_body, 0)
        process(nchunk - 1, 0, True)
        pltpu.make_async_copy(obuf.at[0], out_hbm.at[pl.ds(base, CHUNK)], so0).wait()
        pltpu.make_async_copy(obuf.at[1], out_hbm.at[pl.ds(base, CHUNK)], so1).wait()

    return k(P, Q, sender3d, receiver3d)


def _sc_scatter(M, receiver3d, n_nodes):
    nchunk = receiver3d.shape[1]
    epw = nchunk * CHUNK
    rows_per_sub = (n_nodes // NS) // 8 * 8
    tail_off = NS * rows_per_sub
    tail_rows = n_nodes - tail_off
    zr = 16
    mesh = plsc.VectorSubcoreMesh(core_axis_name="c", subcore_axis_name="s",
                                  num_cores=NC, num_subcores=NS)

    @functools.partial(
        pl.kernel,
        out_type=jax.ShapeDtypeStruct((NC, n_nodes, D), jnp.float32),
        mesh=mesh,
        scratch_types=[
            pltpu.VMEM((4, CHUNK), jnp.int32),
            pltpu.VMEM((4, CHUNK, D), jnp.float32),
            pltpu.VMEM((zr, D), jnp.float32),
            pltpu.VMEM_SHARED((n_nodes, D), jnp.float32),
            pltpu.SemaphoreType.DMA, pltpu.SemaphoreType.DMA,
            pltpu.SemaphoreType.DMA, pltpu.SemaphoreType.DMA,
            pltpu.SemaphoreType.DMA, pltpu.SemaphoreType.DMA,
            pltpu.SemaphoreType.DMA, pltpu.SemaphoreType.DMA,
        ],
    )
    def k(m_hbm, r_hbm, out_hbm, idx, bufm, zbuf, acc,
          sf0, sf1, sf2, sf3, ss0, ss1, ss2, ss3):
        sf = (sf0, sf1, sf2, sf3)
        ss = (ss0, ss1, ss2, ss3)
        cid = lax.axis_index("c")
        sid = lax.axis_index("s")
        wid = sid * NC + cid

        zero = jnp.zeros((16,), jnp.float32)

        def zrow(e, cc):
            for j in range(D // 16):
                zbuf[e, pl.ds(j * 16, 16)] = zero
            return cc

        lax.fori_loop(0, zr, zrow, 0)

        def zcopy(kk, cc):
            pltpu.sync_copy(zbuf, acc.at[pl.ds(sid * rows_per_sub + kk * zr, zr)])
            return cc

        lax.fori_loop(0, rows_per_sub // zr, zcopy, 0)

        @pl.when(sid == NS - 1)
        def _():
            pltpu.sync_copy(zbuf, acc.at[pl.ds(tail_off, tail_rows)])

        plsc.subcore_barrier()

        base = wid * epw

        def fill(c, b):
            off = base + c * CHUNK
            pltpu.async_copy(r_hbm.at[wid, c], idx.at[b], sf[b])
            pltpu.async_copy(m_hbm.at[pl.ds(off, CHUNK)], bufm.at[b], sf[b])

        def wait_fill(c, b):
            off = base + c * CHUNK
            pltpu.make_async_copy(r_hbm.at[wid, c], idx.at[b], sf[b]).wait()
            pltpu.make_async_copy(m_hbm.at[pl.ds(off, CHUNK)], bufm.at[b], sf[b]).wait()

        def wait_scat(b):
            pltpu.make_async_copy(bufm.at[b], acc.at[idx.at[b]], ss[b]).wait()

        fill(0, 0)
        fill(1, 1)

        def ring_body(g4, carry):
            for kk in range(4):
                g = 4 * g4 + kk
                b = kk
                b2 = (kk + 2) % 4

                @pl.when(g < nchunk)
                def _():
                    wait_fill(g, b)
                    pltpu.async_copy(bufm.at[b], acc.at[idx.at[b]], ss[b], add=True)

                @pl.when((g >= 2) & (g - 2 < nchunk))
                def _():
                    wait_scat(b2)

                @pl.when(g + 2 < nchunk)
                def _():
                    fill(g + 2, b2)

            return carry

        lax.fori_loop(0, (nchunk + 3) // 4, ring_body, 0)
        plsc.subcore_barrier()
        pltpu.sync_copy(acc.at[pl.ds(sid * rows_per_sub, rows_per_sub)],
                        out_hbm.at[cid, pl.ds(sid * rows_per_sub, rows_per_sub)])

        @pl.when(sid == NS - 1)
        def _():
            pltpu.sync_copy(acc.at[pl.ds(tail_off, tail_rows)],
                            out_hbm.at[cid, pl.ds(tail_off, tail_rows)])

    return k(M, receiver3d)


def kernel(x_nodes, x_edges, edge_index, We1, be1, We2, be2, Wn1, bn1, Wn2, bn2):
    n_nodes = x_nodes.shape[0]
    n_edges = edge_index.shape[1]
    nchunk = n_edges // NW // CHUNK
    sender = edge_index[0].astype(jnp.int32).reshape(NW, nchunk, CHUNK)
    receiver = edge_index[1].astype(jnp.int32).reshape(NW, nchunk, CHUNK)
    Ws, Wr, We = We1[:D], We1[D:2 * D], We1[2 * D:]
    b1e = be1.reshape(1, D)
    b2e = be2.reshape(1, D)
    b1n = bn1.reshape(1, D)
    b2n = bn2.reshape(1, D)

    P, Q = _compute_pq(x_nodes, Ws, Wr, nb=2000)
    G = _sc_gather(P, Q, sender, receiver)
    M = _edge_mlp(G, x_edges, We, b1e, We2, b2e, eb=2000)
    A = _sc_scatter(M, receiver, n_nodes)
    return _node_net(x_nodes, A, Wn1[:D], Wn1[D:], b1n, Wn2, b2n, nb=2000)
